# baseline (speedup 1.0000x reference)
import functools

import jax
import jax.numpy as jnp
from jax import lax
from jax.experimental import pallas as pl
from jax.experimental.pallas import tpu as pltpu
from jax.experimental.pallas import tpu_sc as plsc

NUM_CORES = 2
NUM_SUBCORES = 16
NUM_WORKERS = NUM_CORES * NUM_SUBCORES

B, N, D = 128, 576, 768
BPW = B // NUM_WORKERS
P = 32
PB = P * D
NBLK = N // P
GROUPS = PB // 16
UNROLL = 8


def _sc_body(patch_hbm, pos_hbm, out_hbm, pos_v, in_v):
    c = lax.axis_index("c")
    s = lax.axis_index("s")
    wid = s * NUM_CORES + c
    b0 = wid * BPW

    def p_loop(p, carry):
        pltpu.sync_copy(pos_hbm.at[pl.ds(p * PB, PB)], pos_v)

        def b_loop(b, carry2):
            bi = b0 + b
            pltpu.sync_copy(patch_hbm.at[bi, pl.ds(p * PB, PB)], in_v)

            def g_loop(g, carry3):
                for u in range(UNROLL):
                    off = (g * UNROLL + u) * 16
                    in_v[pl.ds(off, 16)] = in_v[pl.ds(off, 16)] + pos_v[pl.ds(off, 16)]
                return carry3

            lax.fori_loop(0, GROUPS // UNROLL, g_loop, 0)
            pltpu.sync_copy(in_v, out_hbm.at[bi, pl.ds(p * PB, PB)])
            return carry2

        lax.fori_loop(0, BPW, b_loop, 0)
        return carry

    lax.fori_loop(0, NBLK, p_loop, 0)


def _make_sc_add():
    mesh = plsc.VectorSubcoreMesh(
        core_axis_name="c",
        subcore_axis_name="s",
        num_cores=NUM_CORES,
        num_subcores=NUM_SUBCORES,
    )
    return pl.kernel(
        _sc_body,
        out_type=jax.ShapeDtypeStruct((B, N * D), jnp.float32),
        mesh=mesh,
        scratch_types=[
            pltpu.VMEM((PB,), jnp.float32),
            pltpu.VMEM((PB,), jnp.float32),
        ],
    )


def kernel(patch, pos_table):
    patch2 = patch.reshape(B, N * D)
    pos2 = pos_table.reshape(N * D)
    out = _make_sc_add()(patch2, pos2)
    return out.reshape(B, N, D)


# speedup vs baseline: 1.2392x; 1.2392x over previous
import jax
import jax.numpy as jnp
from jax import lax
from jax.experimental import pallas as pl
from jax.experimental.pallas import tpu as pltpu
from jax.experimental.pallas import tpu_sc as plsc

NUM_CORES = 2
NUM_SUBCORES = 16
NUM_WORKERS = NUM_CORES * NUM_SUBCORES

B, N, D = 128, 576, 768
BPW = B // NUM_WORKERS
P = 48
PB = P * D
NBLK = N // P
TOT = NBLK * BPW
GROUPS = PB // 16
UNROLL = 8


def _sc_body(patch_hbm, pos_hbm, out_hbm, pos_v, buf0, buf1,
             sin0, sin1, sout0, sout1):
    c = lax.axis_index("c")
    s = lax.axis_index("s")
    wid = s * NUM_CORES + c
    b0 = wid * BPW

    bufs = (buf0, buf1)
    sins = (sin0, sin1)
    souts = (sout0, sout1)

    def patch_slice(k):
        p = k // BPW
        b = k % BPW
        return patch_hbm.at[b0 + b, pl.ds(p * PB, PB)]

    def out_slice(k):
        p = k // BPW
        b = k % BPW
        return out_hbm.at[b0 + b, pl.ds(p * PB, PB)]

    pltpu.async_copy(patch_slice(0), bufs[0], sins[0])

    def step(k, q):
        nq = 1 - q

        @pl.when(k >= 1)
        def _():
            pltpu.make_async_copy(bufs[nq], out_slice(k - 1), souts[nq]).wait()

        @pl.when(k + 1 < TOT)
        def _():
            pltpu.async_copy(patch_slice(k + 1), bufs[nq], sins[nq])

        pltpu.make_async_copy(patch_slice(k), bufs[q], sins[q]).wait()

        @pl.when(k % BPW == 0)
        def _():
            pltpu.sync_copy(pos_hbm.at[pl.ds((k // BPW) * PB, PB)], pos_v)

        @plsc.parallel_loop(0, GROUPS, 1, unroll=UNROLL)
        def _(g):
            off = g * 16
            bufs[q][pl.ds(off, 16)] = (
                bufs[q][pl.ds(off, 16)] + pos_v[pl.ds(off, 16)]
            )

        pltpu.async_copy(bufs[q], out_slice(k), souts[q])

    def pair(t, carry):
        step(2 * t, 0)
        step(2 * t + 1, 1)
        return carry

    lax.fori_loop(0, TOT // 2, pair, 0)

    pltpu.make_async_copy(bufs[1], out_slice(TOT - 1), souts[1]).wait()


def _make_sc_add():
    mesh = plsc.VectorSubcoreMesh(
        core_axis_name="c",
        subcore_axis_name="s",
        num_cores=NUM_CORES,
        num_subcores=NUM_SUBCORES,
    )
    return pl.kernel(
        _sc_body,
        out_type=jax.ShapeDtypeStruct((B, N * D), jnp.float32),
        mesh=mesh,
        scratch_types=[
            pltpu.VMEM((PB,), jnp.float32),
            pltpu.VMEM((PB,), jnp.float32),
            pltpu.VMEM((PB,), jnp.float32),
            pltpu.SemaphoreType.DMA,
            pltpu.SemaphoreType.DMA,
            pltpu.SemaphoreType.DMA,
            pltpu.SemaphoreType.DMA,
        ],
    )


def kernel(patch, pos_table):
    patch2 = patch.reshape(B, N * D)
    pos2 = pos_table.reshape(N * D)
    out = _make_sc_add()(patch2, pos2)
    return out.reshape(B, N, D)


# speedup vs baseline: 2.6369x; 2.1280x over previous
import jax
import jax.numpy as jnp
from jax import lax
from jax.experimental import pallas as pl
from jax.experimental.pallas import tpu as pltpu
from jax.experimental.pallas import tpu_sc as plsc

NUM_CORES = 2
NUM_SUBCORES = 16
NUM_WORKERS = NUM_CORES * NUM_SUBCORES

B, N, D = 128, 576, 768
BPW = B // NUM_WORKERS
P = 48
NBLK = N // P
TOT = NBLK * BPW
NGRP = D // 16
ROW_UNROLL = 2


def _sc_body(patch_hbm, pos_hbm, out_hbm, pos_v, buf0, buf1,
             sin0, sin1, sout0, sout1):
    c = lax.axis_index("c")
    s = lax.axis_index("s")
    wid = s * NUM_CORES + c
    b0 = wid * BPW

    bufs = (buf0, buf1)
    sins = (sin0, sin1)
    souts = (sout0, sout1)

    def patch_slice(k):
        p = k // BPW
        b = k % BPW
        return patch_hbm.at[b0 + b, pl.ds(p * P, P), :]

    def out_slice(k):
        p = k // BPW
        b = k % BPW
        return out_hbm.at[b0 + b, pl.ds(p * P, P), :]

    pltpu.async_copy(patch_slice(0), bufs[0], sins[0])

    def step(k, q):
        nq = 1 - q

        @pl.when(k >= 1)
        def _():
            pltpu.make_async_copy(bufs[nq], out_slice(k - 1), souts[nq]).wait()

        @pl.when(k + 1 < TOT)
        def _():
            pltpu.async_copy(patch_slice(k + 1), bufs[nq], sins[nq])

        pltpu.make_async_copy(patch_slice(k), bufs[q], sins[q]).wait()

        @pl.when(k % BPW == 0)
        def _():
            pltpu.sync_copy(pos_hbm.at[pl.ds((k // BPW) * P, P), :], pos_v)

        @plsc.parallel_loop(0, P, ROW_UNROLL, unroll=1)
        def _(r):
            for ru in range(ROW_UNROLL):
                for j in range(NGRP):
                    sl = pl.ds(j * 16, 16)
                    bufs[q][r + ru, sl] = (
                        bufs[q][r + ru, sl] + pos_v[r + ru, sl]
                    )

        pltpu.async_copy(bufs[q], out_slice(k), souts[q])

    def pair(t, carry):
        step(2 * t, 0)
        step(2 * t + 1, 1)
        return carry

    lax.fori_loop(0, TOT // 2, pair, 0)

    pltpu.make_async_copy(bufs[1], out_slice(TOT - 1), souts[1]).wait()


def _make_sc_add():
    mesh = plsc.VectorSubcoreMesh(
        core_axis_name="c",
        subcore_axis_name="s",
        num_cores=NUM_CORES,
        num_subcores=NUM_SUBCORES,
    )
    return pl.kernel(
        _sc_body,
        out_type=jax.ShapeDtypeStruct((B, N, D), jnp.float32),
        mesh=mesh,
        scratch_types=[
            pltpu.VMEM((P, D), jnp.float32),
            pltpu.VMEM((P, D), jnp.float32),
            pltpu.VMEM((P, D), jnp.float32),
            pltpu.SemaphoreType.DMA,
            pltpu.SemaphoreType.DMA,
            pltpu.SemaphoreType.DMA,
            pltpu.SemaphoreType.DMA,
        ],
        compiler_params=pltpu.CompilerParams(use_tc_tiling_on_sc=True),
    )


def kernel(patch, pos_table):
    return _make_sc_add()(patch, pos_table)


# speedup vs baseline: 2.9848x; 1.1319x over previous
import jax
import jax.numpy as jnp
from jax import lax
from jax.experimental import pallas as pl
from jax.experimental.pallas import tpu as pltpu
from jax.experimental.pallas import tpu_sc as plsc

NUM_CORES = 2
NUM_SUBCORES = 16
NUM_WORKERS = NUM_CORES * NUM_SUBCORES

B, N, D = 128, 576, 768
BPW = B // NUM_WORKERS
P = 32
NBLK = N // P
TOT = NBLK * BPW
NGRP = D // 16
CHUNK = 12


def _sc_body(patch_hbm, pos_hbm, out_hbm,
             ring0, ring1, ring2, pos_v,
             sin0, sin1, sin2, sout0, sout1, sout2):
    c = lax.axis_index("c")
    s = lax.axis_index("s")
    wid = s * NUM_CORES + c
    b0 = wid * BPW

    rings = (ring0, ring1, ring2)
    sins = (sin0, sin1, sin2)
    souts = (sout0, sout1, sout2)

    def patch_slice(k):
        return patch_hbm.at[b0 + k % BPW, pl.ds((k // BPW) * P, P), :]

    def out_slice(k):
        return out_hbm.at[b0 + k % BPW, pl.ds((k // BPW) * P, P), :]

    pltpu.async_copy(patch_slice(0), rings[0], sins[0])
    pltpu.async_copy(patch_slice(1), rings[1], sins[1])

    def step(k, i):
        r = i % 3

        if i % BPW == 0:
            pltpu.sync_copy(pos_hbm.at[pl.ds((k // BPW) * P, P), :], pos_v)

        pltpu.make_async_copy(patch_slice(k), rings[r], sins[r]).wait()

        @plsc.parallel_loop(0, P, 1, unroll=1)
        def _(row):
            for j in range(NGRP):
                sl = pl.ds(j * 16, 16)
                rings[r][row, sl] = rings[r][row, sl] + pos_v[row, sl]

        pltpu.async_copy(rings[r], out_slice(k), souts[r])

        nr = (i + 2) % 3

        @pl.when((k >= 1) & (k + 2 < TOT))
        def _():
            pltpu.make_async_copy(rings[nr], out_slice(k - 1), souts[nr]).wait()

        @pl.when(k + 2 < TOT)
        def _():
            pltpu.async_copy(patch_slice(k + 2), rings[nr], sins[nr])

    def chunk(t, carry):
        for i in range(CHUNK):
            step(t * CHUNK + i, i)
        return carry

    lax.fori_loop(0, TOT // CHUNK, chunk, 0)

    for k in (TOT - 3, TOT - 2, TOT - 1):
        pltpu.make_async_copy(rings[k % 3], out_slice(k), souts[k % 3]).wait()


def _make_sc_add():
    mesh = plsc.VectorSubcoreMesh(
        core_axis_name="c",
        subcore_axis_name="s",
        num_cores=NUM_CORES,
        num_subcores=NUM_SUBCORES,
    )
    return pl.kernel(
        _sc_body,
        out_type=jax.ShapeDtypeStruct((B, N, D), jnp.float32),
        mesh=mesh,
        scratch_types=[
            pltpu.VMEM((P, D), jnp.float32),
            pltpu.VMEM((P, D), jnp.float32),
            pltpu.VMEM((P, D), jnp.float32),
            pltpu.VMEM((P, D), jnp.float32),
            pltpu.SemaphoreType.DMA,
            pltpu.SemaphoreType.DMA,
            pltpu.SemaphoreType.DMA,
            pltpu.SemaphoreType.DMA,
            pltpu.SemaphoreType.DMA,
            pltpu.SemaphoreType.DMA,
        ],
        compiler_params=pltpu.CompilerParams(use_tc_tiling_on_sc=True),
    )


def kernel(patch, pos_table):
    return _make_sc_add()(patch, pos_table)


# speedup vs baseline: 3.8440x; 1.2879x over previous
import jax
import jax.numpy as jnp
from jax import lax
from jax.experimental import pallas as pl
from jax.experimental.pallas import tpu as pltpu
from jax.experimental.pallas import tpu_sc as plsc

NUM_CORES = 2
NUM_SUBCORES = 16
NUM_WORKERS = NUM_CORES * NUM_SUBCORES

B, N, D = 128, 576, 768
B_SC = 16
B_TC = B - B_SC
SLABS_PER_BATCH = NUM_WORKERS // B_SC
ROWS = N // SLABS_PER_BATCH
P = 32
NBLK = ROWS // P
NGRP = D // 16
BATCH_BLOCK = 8
CHUNK = 3


def _tc_add_body(patch_ref, pos_ref, out_ref):
    out_ref[...] = patch_ref[...] + pos_ref[...][None, :, :]


def _tc_add(patch, pos_table):
    return pl.pallas_call(
        _tc_add_body,
        grid=(B_TC // BATCH_BLOCK,),
        in_specs=[
            pl.BlockSpec((BATCH_BLOCK, N, D), lambda i: (i, 0, 0)),
            pl.BlockSpec((N, D), lambda i: (0, 0)),
        ],
        out_specs=pl.BlockSpec((BATCH_BLOCK, N, D), lambda i: (i, 0, 0)),
        out_shape=jax.ShapeDtypeStruct((B, N, D), patch.dtype),
    )(patch, pos_table)


def _sc_body(patch_hbm, pos_hbm, out_hbm,
             ring0, ring1, ring2, pos_v,
             sin0, sin1, sin2, sout0, sout1, sout2):
    c = lax.axis_index("c")
    s = lax.axis_index("s")
    wid = s * NUM_CORES + c
    bi = B_TC + wid // SLABS_PER_BATCH
    half = wid % SLABS_PER_BATCH
    row0 = half * ROWS

    rings = (ring0, ring1, ring2)
    sins = (sin0, sin1, sin2)
    souts = (sout0, sout1, sout2)

    def patch_slice(k):
        return patch_hbm.at[bi, pl.ds(row0 + k * P, P), :]

    def out_slice(k):
        return out_hbm.at[bi - B_TC, pl.ds(row0 + k * P, P), :]

    def pos_slice(k):
        return pos_hbm.at[pl.ds(row0 + k * P, P), :]

    pltpu.async_copy(patch_slice(0), rings[0], sins[0])
    pltpu.async_copy(patch_slice(1), rings[1], sins[1])

    def step(k, i):
        r = i % 3

        pltpu.sync_copy(pos_slice(k), pos_v)
        pltpu.make_async_copy(patch_slice(k), rings[r], sins[r]).wait()

        @plsc.parallel_loop(0, P, 1, unroll=1)
        def _(row):
            for j in range(NGRP):
                sl = pl.ds(j * 16, 16)
                rings[r][row, sl] = rings[r][row, sl] + pos_v[row, sl]

        pltpu.async_copy(rings[r], out_slice(k), souts[r])

        nr = (i + 2) % 3

        @pl.when((k >= 1) & (k + 2 < NBLK))
        def _():
            pltpu.make_async_copy(rings[nr], out_slice(k - 1), souts[nr]).wait()

        @pl.when(k + 2 < NBLK)
        def _():
            pltpu.async_copy(patch_slice(k + 2), rings[nr], sins[nr])

    def chunk(t, carry):
        for i in range(CHUNK):
            step(t * CHUNK + i, i)
        return carry

    lax.fori_loop(0, NBLK // CHUNK, chunk, 0)

    for k in (NBLK - 3, NBLK - 2, NBLK - 1):
        pltpu.make_async_copy(rings[k % 3], out_slice(k), souts[k % 3]).wait()


def _make_sc_add():
    mesh = plsc.VectorSubcoreMesh(
        core_axis_name="c",
        subcore_axis_name="s",
        num_cores=NUM_CORES,
        num_subcores=NUM_SUBCORES,
    )
    return pl.kernel(
        _sc_body,
        out_type=jax.ShapeDtypeStruct((B_SC, N, D), jnp.float32),
        mesh=mesh,
        scratch_types=[
            pltpu.VMEM((P, D), jnp.float32),
            pltpu.VMEM((P, D), jnp.float32),
            pltpu.VMEM((P, D), jnp.float32),
            pltpu.VMEM((P, D), jnp.float32),
            pltpu.SemaphoreType.DMA,
            pltpu.SemaphoreType.DMA,
            pltpu.SemaphoreType.DMA,
            pltpu.SemaphoreType.DMA,
            pltpu.SemaphoreType.DMA,
            pltpu.SemaphoreType.DMA,
        ],
        compiler_params=pltpu.CompilerParams(use_tc_tiling_on_sc=True),
    )


def kernel(patch, pos_table):
    sc_out = _make_sc_add()(patch, pos_table)
    tc_out = _tc_add(patch, pos_table)
    return lax.dynamic_update_slice(tc_out, sc_out, (B_TC, 0, 0))


# speedup vs baseline: 4.4130x; 1.1480x over previous
import jax
import jax.numpy as jnp
from jax import lax
from jax.experimental import pallas as pl
from jax.experimental.pallas import tpu as pltpu
from jax.experimental.pallas import tpu_sc as plsc

NUM_CORES = 2
NUM_SUBCORES = 16
NUM_WORKERS = NUM_CORES * NUM_SUBCORES

B, N, D = 128, 576, 768
B_SC = 4
B_TC = B - B_SC
SLABS_PER_BATCH = NUM_WORKERS // B_SC
ROWS = N // SLABS_PER_BATCH
P = 24
NBLK = ROWS // P
NGRP = D // 16
BATCH_BLOCK = 4
CHUNK = 3


def _tc_add_body(patch_ref, pos_ref, out_ref):
    out_ref[...] = patch_ref[...] + pos_ref[...][None, :, :]


def _tc_add(patch, pos_table):
    return pl.pallas_call(
        _tc_add_body,
        grid=(B_TC // BATCH_BLOCK,),
        in_specs=[
            pl.BlockSpec((BATCH_BLOCK, N, D), lambda i: (i, 0, 0)),
            pl.BlockSpec((N, D), lambda i: (0, 0)),
        ],
        out_specs=pl.BlockSpec((BATCH_BLOCK, N, D), lambda i: (i, 0, 0)),
        out_shape=jax.ShapeDtypeStruct((B, N, D), patch.dtype),
    )(patch, pos_table)


def _sc_body(patch_hbm, pos_hbm, out_hbm,
             ring0, ring1, ring2, pos_v,
             sin0, sin1, sin2, sout0, sout1, sout2):
    c = lax.axis_index("c")
    s = lax.axis_index("s")
    wid = s * NUM_CORES + c
    bi = B_TC + wid // SLABS_PER_BATCH
    half = wid % SLABS_PER_BATCH
    row0 = half * ROWS

    rings = (ring0, ring1, ring2)
    sins = (sin0, sin1, sin2)
    souts = (sout0, sout1, sout2)

    def patch_slice(k):
        return patch_hbm.at[bi, pl.ds(row0 + k * P, P), :]

    def out_slice(k):
        return out_hbm.at[bi - B_TC, pl.ds(row0 + k * P, P), :]

    def pos_slice(k):
        return pos_hbm.at[pl.ds(row0 + k * P, P), :]

    pltpu.async_copy(patch_slice(0), rings[0], sins[0])
    pltpu.async_copy(patch_slice(1), rings[1], sins[1])

    def step(k, i):
        r = i % 3

        pltpu.sync_copy(pos_slice(k), pos_v)
        pltpu.make_async_copy(patch_slice(k), rings[r], sins[r]).wait()

        @plsc.parallel_loop(0, P, 1, unroll=1)
        def _(row):
            for j in range(NGRP):
                sl = pl.ds(j * 16, 16)
                rings[r][row, sl] = rings[r][row, sl] + pos_v[row, sl]

        pltpu.async_copy(rings[r], out_slice(k), souts[r])

        nr = (i + 2) % 3

        @pl.when((k >= 1) & (k + 2 < NBLK))
        def _():
            pltpu.make_async_copy(rings[nr], out_slice(k - 1), souts[nr]).wait()

        @pl.when(k + 2 < NBLK)
        def _():
            pltpu.async_copy(patch_slice(k + 2), rings[nr], sins[nr])

    def chunk(t, carry):
        for i in range(CHUNK):
            step(t * CHUNK + i, i)
        return carry

    lax.fori_loop(0, NBLK // CHUNK, chunk, 0)

    for k in (NBLK - 3, NBLK - 2, NBLK - 1):
        pltpu.make_async_copy(rings[k % 3], out_slice(k), souts[k % 3]).wait()


def _make_sc_add():
    mesh = plsc.VectorSubcoreMesh(
        core_axis_name="c",
        subcore_axis_name="s",
        num_cores=NUM_CORES,
        num_subcores=NUM_SUBCORES,
    )
    return pl.kernel(
        _sc_body,
        out_type=jax.ShapeDtypeStruct((B_SC, N, D), jnp.float32),
        mesh=mesh,
        scratch_types=[
            pltpu.VMEM((P, D), jnp.float32),
            pltpu.VMEM((P, D), jnp.float32),
            pltpu.VMEM((P, D), jnp.float32),
            pltpu.VMEM((P, D), jnp.float32),
            pltpu.SemaphoreType.DMA,
            pltpu.SemaphoreType.DMA,
            pltpu.SemaphoreType.DMA,
            pltpu.SemaphoreType.DMA,
            pltpu.SemaphoreType.DMA,
            pltpu.SemaphoreType.DMA,
        ],
        compiler_params=pltpu.CompilerParams(use_tc_tiling_on_sc=True),
    )


def kernel(patch, pos_table):
    sc_out = _make_sc_add()(patch, pos_table)
    tc_out = _tc_add(patch, pos_table)
    return lax.dynamic_update_slice(tc_out, sc_out, (B_TC, 0, 0))
